# baseline (speedup 1.0000x reference)
import functools

import jax
import jax.numpy as jnp
import numpy as np
from jax import lax
from jax.experimental import pallas as pl
from jax.experimental.pallas import tpu as pltpu
from jax.experimental.pallas import tpu_sc as plsc

N_ENTITIES = 1000000
N_RELATIONS = 1000
K = 64
BATCH = 16384

NC = 2
NS = 16
NW = NC * NS
B_PER_W = BATCH // NW
CHUNK = 128
N_CHUNKS = B_PER_W // CHUNK
GROUPS = CHUNK // 16

_LANE = np.arange(16, dtype=np.int32)


_GATHER_DNUMS = lax.GatherDimensionNumbers(
    offset_dims=(), collapsed_slice_dims=(0,), start_index_map=(0,)
)


def _shuffle(v, perm):
    return lax.gather(
        v, perm[:, None], _GATHER_DNUMS, (1,),
        mode=lax.GatherScatterMode.PROMISE_IN_BOUNDS,
    )


def _hsum(v, lane):
    for s in (8, 4, 2, 1):
        v = v + _shuffle(v, lane ^ s)
    return v


def _sqrt16(a):
    a = jnp.maximum(a, jnp.float32(1e-30))
    bits = lax.bitcast_convert_type(a, jnp.int32)
    y = lax.bitcast_convert_type(
        jnp.int32(0x5F3759DF) - lax.shift_right_logical(bits, 1), jnp.float32
    )
    half = jnp.float32(0.5) * a
    for _ in range(3):
        y = y * (jnp.float32(1.5) - half * y * y)
    return a * y


def _body(head_hbm, rel_hbm, tail_hbm, ent_hbm, rel_emb_hbm, norm_emb_hbm,
          out_hbm, hidx, ridx, tidx, hrows, trows, rrows, nrows, loss_v,
          s0, s1, s2, s3):
    wid = lax.axis_index("s") * NC + lax.axis_index("c")
    pltpu.sync_copy(head_hbm.at[pl.ds(wid * N_CHUNKS, N_CHUNKS)], hidx)
    pltpu.sync_copy(rel_hbm.at[pl.ds(wid * N_CHUNKS, N_CHUNKS)], ridx)
    pltpu.sync_copy(tail_hbm.at[pl.ds(wid * N_CHUNKS, N_CHUNKS)], tidx)

    for c in range(N_CHUNKS):
        cp0 = pltpu.async_copy(ent_hbm.at[hidx.at[c]], hrows, s0)
        cp1 = pltpu.async_copy(ent_hbm.at[tidx.at[c]], trows, s1)
        cp2 = pltpu.async_copy(rel_emb_hbm.at[ridx.at[c]], rrows, s2)
        cp3 = pltpu.async_copy(norm_emb_hbm.at[ridx.at[c]], nrows, s3)
        cp0.wait()
        cp1.wait()
        cp2.wait()
        cp3.wait()

        def group(g, _):
            lane = lax.iota(jnp.int32, 16)
            acc = jnp.zeros((16,), jnp.float32)
            base = g * 16
            for i in range(16):
                row = base + i
                h = [hrows[row, pl.ds(16 * j, 16)] for j in range(4)]
                t = [trows[row, pl.ds(16 * j, 16)] for j in range(4)]
                r = [rrows[row, pl.ds(16 * j, 16)] for j in range(4)]
                n = [nrows[row, pl.ds(16 * j, 16)] for j in range(4)]
                u = [h[j] - t[j] for j in range(4)]
                p = u[0] * n[0] + u[1] * n[1] + u[2] * n[2] + u[3] * n[3]
                d = _hsum(p, lane)
                df = [u[j] + r[j] - d * n[j] for j in range(4)]
                sq = df[0] * df[0] + df[1] * df[1] + df[2] * df[2] + df[3] * df[3]
                ss = _hsum(sq, lane)
                acc = jnp.where(lane == i, ss, acc)
            loss_v[pl.ds(c * CHUNK + g * 16, 16)] = _sqrt16(acc)
            return _

        lax.fori_loop(0, GROUPS, group, 0)

    pltpu.sync_copy(loss_v, out_hbm.at[pl.ds(wid * B_PER_W, B_PER_W)])


@functools.partial(jax.jit, static_argnames=())
def _transh(head2d, rel2d, tail2d, entity_emb, relation_emb, norm_emb):
    mesh = plsc.VectorSubcoreMesh(core_axis_name="c", subcore_axis_name="s")
    kfn = pl.kernel(
        _body,
        out_type=jax.ShapeDtypeStruct((BATCH,), jnp.float32),
        mesh=mesh,
        scratch_types=[
            pltpu.VMEM((N_CHUNKS, CHUNK), jnp.int32),
            pltpu.VMEM((N_CHUNKS, CHUNK), jnp.int32),
            pltpu.VMEM((N_CHUNKS, CHUNK), jnp.int32),
            pltpu.VMEM((CHUNK, K), jnp.float32),
            pltpu.VMEM((CHUNK, K), jnp.float32),
            pltpu.VMEM((CHUNK, K), jnp.float32),
            pltpu.VMEM((CHUNK, K), jnp.float32),
            pltpu.VMEM((B_PER_W,), jnp.float32),
            pltpu.SemaphoreType.DMA,
            pltpu.SemaphoreType.DMA,
            pltpu.SemaphoreType.DMA,
            pltpu.SemaphoreType.DMA,
        ],
        compiler_params=pltpu.CompilerParams(use_tc_tiling_on_sc=False),
    )
    return kfn(head2d, rel2d, tail2d, entity_emb, relation_emb, norm_emb)


def kernel(head, relation, tail, entity_emb, relation_emb, norm_emb):
    head2d = jnp.asarray(head, jnp.int32).reshape(NW * N_CHUNKS, CHUNK)
    rel2d = jnp.asarray(relation, jnp.int32).reshape(NW * N_CHUNKS, CHUNK)
    tail2d = jnp.asarray(tail, jnp.int32).reshape(NW * N_CHUNKS, CHUNK)
    return _transh(head2d, rel2d, tail2d, entity_emb, relation_emb, norm_emb)


# speedup vs baseline: 1.0082x; 1.0082x over previous
import functools

import jax
import jax.numpy as jnp
import numpy as np
from jax import lax
from jax.experimental import pallas as pl
from jax.experimental.pallas import tpu as pltpu
from jax.experimental.pallas import tpu_sc as plsc

N_ENTITIES = 1000000
N_RELATIONS = 1000
K = 64
BATCH = 16384

NC = 2
NS = 16
NW = NC * NS
B_PER_W = BATCH // NW
CHUNK = 128
N_CHUNKS = B_PER_W // CHUNK
GROUPS = CHUNK // 16

_LANE = np.arange(16, dtype=np.int32)

_GATHER_DNUMS = lax.GatherDimensionNumbers(
    offset_dims=(), collapsed_slice_dims=(0,), start_index_map=(0,)
)


def _shuffle(v, perm):
    return lax.gather(
        v, perm[:, None], _GATHER_DNUMS, (1,),
        mode=lax.GatherScatterMode.PROMISE_IN_BOUNDS,
    )


def _hsum(v, lane):
    for s in (8, 4, 2, 1):
        v = v + _shuffle(v, lane ^ s)
    return v


def _sqrt16(a):
    a = jnp.maximum(a, jnp.float32(1e-30))
    bits = lax.bitcast_convert_type(a, jnp.int32)
    y = lax.bitcast_convert_type(
        jnp.int32(0x5F3759DF) - lax.shift_right_logical(bits, 1), jnp.float32
    )
    half = jnp.float32(0.5) * a
    for _ in range(3):
        y = y * (jnp.float32(1.5) - half * y * y)
    return a * y


def _body(idx_hbm, ent_hbm, rn_hbm, out_hbm,
          idx_v, h0, t0, rn0, h1, t1, rn1, loss_v,
          sa0, sb0, sc0, sa1, sb1, sc1):
    wid = lax.axis_index("s") * NC + lax.axis_index("c")
    pltpu.sync_copy(idx_hbm.at[wid], idx_v)

    bufs = ((h0, t0, rn0, sa0, sb0, sc0), (h1, t1, rn1, sa1, sb1, sc1))

    def fire(c, b):
        h, t, rn, sa, sb, sc = b
        return (
            pltpu.async_copy(ent_hbm.at[idx_v.at[c]], h, sa),
            pltpu.async_copy(ent_hbm.at[idx_v.at[N_CHUNKS + c]], t, sb),
            pltpu.async_copy(rn_hbm.at[idx_v.at[2 * N_CHUNKS + c]], rn, sc),
        )

    cps = fire(0, bufs[0])
    for c in range(N_CHUNKS):
        ncps = fire(c + 1, bufs[(c + 1) % 2]) if c + 1 < N_CHUNKS else None
        for cp in cps:
            cp.wait()
        hrows, trows, rnrows = bufs[c % 2][:3]

        def group(g, _):
            lane = lax.iota(jnp.int32, 16)
            acc = jnp.zeros((16,), jnp.float32)
            base = g * 16
            for i in range(16):
                row = base + i
                h = [hrows[row, pl.ds(16 * j, 16)] for j in range(4)]
                t = [trows[row, pl.ds(16 * j, 16)] for j in range(4)]
                r = [rnrows[row, pl.ds(16 * j, 16)] for j in range(4)]
                n = [rnrows[row, pl.ds(64 + 16 * j, 16)] for j in range(4)]
                u = [h[j] - t[j] for j in range(4)]
                p = u[0] * n[0] + u[1] * n[1] + u[2] * n[2] + u[3] * n[3]
                d = _hsum(p, lane)
                df = [u[j] + r[j] - d * n[j] for j in range(4)]
                sq = df[0] * df[0] + df[1] * df[1] + df[2] * df[2] + df[3] * df[3]
                ss = _hsum(sq, lane)
                acc = jnp.where(lane == i, ss, acc)
            loss_v[pl.ds(c * CHUNK + g * 16, 16)] = _sqrt16(acc)
            return _

        lax.fori_loop(0, GROUPS, group, 0)
        cps = ncps

    pltpu.sync_copy(loss_v, out_hbm.at[pl.ds(wid * B_PER_W, B_PER_W)])


@jax.jit
def _transh(idx_pack, entity_emb, rel_norm):
    mesh = plsc.VectorSubcoreMesh(core_axis_name="c", subcore_axis_name="s")
    kfn = pl.kernel(
        _body,
        out_type=jax.ShapeDtypeStruct((BATCH,), jnp.float32),
        mesh=mesh,
        scratch_types=[
            pltpu.VMEM((3 * N_CHUNKS, CHUNK), jnp.int32),
            pltpu.VMEM((CHUNK, K), jnp.float32),
            pltpu.VMEM((CHUNK, K), jnp.float32),
            pltpu.VMEM((CHUNK, 2 * K), jnp.float32),
            pltpu.VMEM((CHUNK, K), jnp.float32),
            pltpu.VMEM((CHUNK, K), jnp.float32),
            pltpu.VMEM((CHUNK, 2 * K), jnp.float32),
            pltpu.VMEM((B_PER_W,), jnp.float32),
            pltpu.SemaphoreType.DMA,
            pltpu.SemaphoreType.DMA,
            pltpu.SemaphoreType.DMA,
            pltpu.SemaphoreType.DMA,
            pltpu.SemaphoreType.DMA,
            pltpu.SemaphoreType.DMA,
        ],
        compiler_params=pltpu.CompilerParams(use_tc_tiling_on_sc=False),
    )
    return kfn(idx_pack, entity_emb, rel_norm)


def kernel(head, relation, tail, entity_emb, relation_emb, norm_emb):
    h = jnp.asarray(head, jnp.int32).reshape(NW, N_CHUNKS, CHUNK)
    t = jnp.asarray(tail, jnp.int32).reshape(NW, N_CHUNKS, CHUNK)
    r = jnp.asarray(relation, jnp.int32).reshape(NW, N_CHUNKS, CHUNK)
    idx_pack = jnp.concatenate([h, t, r], axis=1)
    rel_norm = jnp.concatenate([relation_emb, norm_emb], axis=1)
    return _transh(idx_pack, entity_emb, rel_norm)
